# Initial kernel scaffold; baseline (speedup 1.0000x reference)
import jax
import jax.numpy as jnp
from jax.experimental import pallas as pl


def kernel(ndata, edata, edge_index, params):
    raise NotImplementedError("write your pallas kernel here")


# speedup vs baseline: 2.9367x; 2.9367x over previous
import functools

import jax
import jax.numpy as jnp
from jax import lax
from jax.experimental import pallas as pl
from jax.experimental.pallas import tpu as pltpu
from jax.experimental.pallas import tpu_sc as plsc

_N = 10000
_E = 320000
_DL = 16
_DNODE_IN = 128
_DNODE_OUT = 128

_NW = 32
_EW = _E // _NW
_C = 80
_K = 5
_G = _C * _K
_NG = _EW // _G
_NZ = _N // 16

_EBLK = 6400
_NBLK = 2000

_SC_MESH = plsc.VectorSubcoreMesh(
    core_axis_name="c", subcore_axis_name="s", num_cores=2, num_subcores=16)

_f32 = jnp.float32


def _ln_tc(x, g, b):
    m = jnp.mean(x, axis=-1, keepdims=True)
    xc = x - m
    v = jnp.mean(xc * xc, axis=-1, keepdims=True)
    return xc * lax.rsqrt(v + 1e-5) * g + b


def _mm(x, w):
    return jnp.dot(x, w, preferred_element_type=_f32)


def _enc_node_body(nd, w1, b1, w2, b2, g, b, w1a, w1b, n_o, p_o, q_o):
    h = jnp.maximum(_mm(nd[...], w1[...]) + b1[...], 0.0)
    nn = _ln_tc(_mm(h, w2[...]) + b2[...], g[...], b[...])
    n_o[...] = nn
    p_o[...] = _mm(nn, w1a[...])
    q_o[...] = _mm(nn, w1b[...])


def _enc_node_call(ndata, enc_w, w1a, w1b):
    grid = (_N // _NBLK,)
    full = lambda a: pl.BlockSpec(a.shape, lambda i: (0,) * a.ndim)
    in_specs = [pl.BlockSpec((_NBLK, _DNODE_IN), lambda i: (i, 0))]
    in_specs += [full(a) for a in enc_w] + [full(w1a), full(w1b)]
    out_specs = [pl.BlockSpec((_NBLK, _DL), lambda i: (i, 0))] * 3
    sd = jax.ShapeDtypeStruct((_N, _DL), _f32)
    return pl.pallas_call(
        _enc_node_body, grid=grid, in_specs=in_specs, out_specs=out_specs,
        out_shape=(sd, sd, sd))(ndata, *enc_w, w1a, w1b)


def _edge_body(has_enc, has_dec, *refs):
    it = iter(refs)
    x = next(it)
    gs = next(it)
    gd = next(it)
    if has_enc:
        w1e, b1e, w2e, b2e, ge, be = (next(it) for _ in range(6))
    w1c, b1, w2, b2, g1, bb1 = (next(it) for _ in range(6))
    if has_dec:
        wd1, bd1, wd2, bd2 = (next(it) for _ in range(4))
    e_o = next(it)
    if has_dec:
        eo_o = next(it)
    e = x[...]
    if has_enc:
        h = jnp.maximum(_mm(e, w1e[...]) + b1e[...], 0.0)
        e = _ln_tc(_mm(h, w2e[...]) + b2e[...], ge[...], be[...])
    h1 = jnp.maximum(gs[...] + gd[...] + _mm(e, w1c[...]) + b1[...], 0.0)
    e1 = e + _ln_tc(_mm(h1, w2[...]) + b2[...], g1[...], bb1[...])
    e_o[...] = e1
    if has_dec:
        hd = jnp.maximum(_mm(e1, wd1[...]) + bd1[...], 0.0)
        eo_o[...] = _mm(hd, wd2[...]) + bd2[...]


def _edge_call(x, gs, gd, enc_w, step_w, dec_w):
    grid = (_E // _EBLK,)
    eb = pl.BlockSpec((_EBLK, _DL), lambda i: (i, 0))
    full = lambda a: pl.BlockSpec(a.shape, lambda i: (0,) * a.ndim)
    weights = list(enc_w or ()) + list(step_w) + list(dec_w or ())
    in_specs = [eb, eb, eb] + [full(a) for a in weights]
    sd = jax.ShapeDtypeStruct((_E, _DL), _f32)
    if dec_w is None:
        out_specs, out_shape = eb, sd
    else:
        out_specs, out_shape = [eb, eb], (sd, sd)
    body = functools.partial(_edge_body, enc_w is not None, dec_w is not None)
    return pl.pallas_call(
        body, grid=grid, in_specs=in_specs, out_specs=out_specs,
        out_shape=out_shape)(x, gs, gd, *weights)


def _node_body(has_dec, *refs):
    it = iter(refs)
    n, pa, pb = next(it), next(it), next(it)
    wn1a, wn1b, bn1, wn2, bn2, gn, bn = (next(it) for _ in range(7))
    if has_dec:
        wd1, bd1, wd2, bd2 = (next(it) for _ in range(4))
        no_o = next(it)
    else:
        w1a, w1b = next(it), next(it)
        n_o, p_o, q_o = next(it), next(it), next(it)
    nb = n[...]
    agg = pa[...] + pb[...]
    h = jnp.maximum(_mm(nb, wn1a[...]) + _mm(agg, wn1b[...]) + bn1[...], 0.0)
    nn = nb + _ln_tc(_mm(h, wn2[...]) + bn2[...], gn[...], bn[...])
    if has_dec:
        hd = jnp.maximum(_mm(nn, wd1[...]) + bd1[...], 0.0)
        no_o[...] = _mm(hd, wd2[...]) + bd2[...]
    else:
        n_o[...] = nn
        p_o[...] = _mm(nn, w1a[...])
        q_o[...] = _mm(nn, w1b[...])


def _node_call(n, partials, step_w, next_pq=None, dec_w=None):
    grid = (_N // _NBLK,)
    nb = pl.BlockSpec((_NBLK, _DL), lambda i: (i, 0))
    pb = pl.BlockSpec((_NBLK, _DL), lambda i: (i + _N // _NBLK, 0))
    full = lambda a: pl.BlockSpec(a.shape, lambda i: (0,) * a.ndim)
    weights = list(step_w) + list(dec_w or ()) + list(next_pq or ())
    in_specs = [nb, nb, pb] + [full(a) for a in weights]
    sd = jax.ShapeDtypeStruct((_N, _DL), _f32)
    if dec_w is None:
        out_specs = [nb, nb, nb]
        out_shape = (sd, sd, sd)
    else:
        out_specs = pl.BlockSpec((_NBLK, _DNODE_OUT), lambda i: (i, 0))
        out_shape = jax.ShapeDtypeStruct((_N, _DNODE_OUT), _f32)
    body = functools.partial(_node_body, dec_w is not None)
    return pl.pallas_call(
        body, grid=grid, in_specs=in_specs, out_specs=out_specs,
        out_shape=out_shape)(n, partials, partials, *weights)


@functools.partial(
    pl.kernel,
    out_type=(jax.ShapeDtypeStruct((_E, _DL), _f32),
              jax.ShapeDtypeStruct((_E, _DL), _f32)),
    mesh=_SC_MESH,
    scratch_types=(
        pltpu.VMEM((_G,), jnp.int32),
        pltpu.VMEM((_G,), jnp.int32),
        pltpu.VMEM((_G, _DL), _f32),
        pltpu.VMEM((_G, _DL), _f32),
        pltpu.SemaphoreType.DMA,
        pltpu.SemaphoreType.DMA,
    ),
    compiler_params=pltpu.CompilerParams(use_tc_tiling_on_sc=False),
)
def _sc_gather(p_hbm, q_hbm, src_hbm, dst_hbm, gs_hbm, gd_hbm,
               isrc, idst, rs, rd, semi, semg):
    wid = lax.axis_index("s") * 2 + lax.axis_index("c")
    base = wid * _EW

    def body(g, carry):
        off = base + g * _G
        a = pltpu.async_copy(src_hbm.at[pl.ds(off, _G)], isrc, semi)
        b = pltpu.async_copy(dst_hbm.at[pl.ds(off, _G)], idst, semi)
        a.wait()
        b.wait()
        descs = []
        for j in range(_K):
            sl = pl.ds(j * _C, _C)
            descs.append(pltpu.async_copy(p_hbm.at[isrc.at[sl]], rs.at[sl], semg))
            descs.append(pltpu.async_copy(q_hbm.at[idst.at[sl]], rd.at[sl], semg))
        for d in descs:
            d.wait()
        pltpu.sync_copy(rs, gs_hbm.at[pl.ds(off, _G)])
        pltpu.sync_copy(rd, gd_hbm.at[pl.ds(off, _G)])
        return carry

    lax.fori_loop(0, _NG, body, 0)


@functools.partial(
    pl.kernel,
    out_type=jax.ShapeDtypeStruct((2 * _N, _DL), _f32),
    mesh=_SC_MESH,
    scratch_types=(
        pltpu.VMEM((_C,), jnp.int32),
        pltpu.VMEM((_C,), jnp.int32),
        pltpu.VMEM((_C,), jnp.int32),
        pltpu.VMEM((_C,), jnp.int32),
        pltpu.VMEM((_C,), jnp.int32),
        pltpu.VMEM((_G, _DL), _f32),
        pltpu.VMEM((_NZ, _DL), _f32),
        pltpu.VMEM_SHARED((_N, _DL), _f32),
        pltpu.SemaphoreType.DMA,
        pltpu.SemaphoreType.DMA,
    ),
    compiler_params=pltpu.CompilerParams(use_tc_tiling_on_sc=False),
)
def _sc_scatter(e_hbm, dst_hbm, out_hbm,
                i0, i1, i2, i3, i4, rows, zbuf, aggs, semi, semr):
    c = lax.axis_index("c")
    s = lax.axis_index("s")
    wid = s * 2 + c
    base = wid * _EW
    idxs = (i0, i1, i2, i3, i4)

    def zr(i, carry):
        zbuf[i, :] = jnp.zeros((_DL,), _f32)
        return carry

    lax.fori_loop(0, _NZ, zr, 0)
    pltpu.sync_copy(zbuf, aggs.at[pl.ds(s * _NZ, _NZ)])
    plsc.subcore_barrier()

    def body(g, carry):
        off = base + g * _G
        descs = [pltpu.async_copy(dst_hbm.at[pl.ds(off + j * _C, _C)],
                                  idxs[j], semi) for j in range(_K)]
        r = pltpu.async_copy(e_hbm.at[pl.ds(off, _G)], rows, semr)
        for d in descs:
            d.wait()
        r.wait()
        sdescs = [pltpu.async_copy(rows.at[pl.ds(j * _C, _C)],
                                   aggs.at[idxs[j]], semr, add=True)
                  for j in range(_K)]
        for d in sdescs:
            d.wait()
        return carry

    lax.fori_loop(0, _NG, body, 0)
    plsc.subcore_barrier()
    pltpu.sync_copy(aggs.at[pl.ds(s * _NZ, _NZ)],
                    out_hbm.at[pl.ds(c * _N + s * _NZ, _NZ)])


def _r2(v):
    return v.reshape(1, -1)


def kernel(ndata, edata, edge_index, params):
    src = edge_index[0].astype(jnp.int32)
    dst = edge_index[1].astype(jnp.int32)

    enc_n = params['enc_node']
    (wn1, bn1), (wn2, bn2) = enc_n['mlp']
    gn, bn = enc_n['ln']
    enc_node_w = (wn1, _r2(bn1), wn2, _r2(bn2), _r2(gn), _r2(bn))

    enc_e = params['enc_edge']
    (we1, be1), (we2, be2) = enc_e['mlp']
    ge, be = enc_e['ln']
    enc_edge_w = (we1, _r2(be1), we2, _r2(be2), _r2(ge), _r2(be))

    steps = []
    for lp in params['proc']:
        (w1, b1), (w2, b2) = lp['edge_mlp']
        g1, bb1 = lp['edge_ln']
        (wn1s, bn1s), (wn2s, bn2s) = lp['node_mlp']
        gns, bns = lp['node_ln']
        steps.append(dict(
            w1a=w1[:_DL], w1b=w1[_DL:2 * _DL],
            edge_w=(w1[2 * _DL:], _r2(b1), w2, _r2(b2), _r2(g1), _r2(bb1)),
            node_w=(wn1s[:_DL], wn1s[_DL:], _r2(bn1s), wn2s, _r2(bn2s),
                    _r2(gns), _r2(bns)),
        ))

    (wdn1, bdn1), (wdn2, bdn2) = params['dec_node']['mlp']
    dec_node_w = (wdn1, _r2(bdn1), wdn2, _r2(bdn2))
    (wde1, bde1), (wde2, bde2) = params['dec_edge']['mlp']
    dec_edge_w = (wde1, _r2(bde1), wde2, _r2(bde2))

    n, p, q = _enc_node_call(ndata, enc_node_w, steps[0]['w1a'], steps[0]['w1b'])

    e = edata
    n_out = e_out = None
    for i in range(3):
        gs, gd = _sc_gather(p, q, src, dst)
        if i == 0:
            e = _edge_call(e, gs, gd, enc_edge_w, steps[0]['edge_w'], None)
        elif i == 1:
            e = _edge_call(e, gs, gd, None, steps[1]['edge_w'], None)
        else:
            e, e_out = _edge_call(e, gs, gd, None, steps[2]['edge_w'], dec_edge_w)
        partials = _sc_scatter(e, dst)
        if i < 2:
            n, p, q = _node_call(n, partials, steps[i]['node_w'],
                                 next_pq=(steps[i + 1]['w1a'], steps[i + 1]['w1b']))
        else:
            n_out = _node_call(n, partials, steps[2]['node_w'], dec_w=dec_node_w)

    return (n_out, e_out)
